# baseline (speedup 1.0000x reference)
import functools

import jax
import jax.numpy as jnp
from jax import lax
from jax.experimental import pallas as pl
from jax.experimental.pallas import tpu as pltpu
from jax.experimental.pallas import tpu_sc as plsc

N = 10000
E = 320000
D_IN = 128
H1 = 128
H2 = 64
OUT = 2
OUT_PAD = 16
EPS = 1e-5

NC = 2
NS = 16
NW = NC * NS
CH = 128
CPT = 80
HCPT = 40
E_PAD = CPT * NW * CH
PAD_ROWS = 240
NP = N + PAD_ROWS
SPT = NP // NS
RPT = 632
NA = NS * RPT


def _segsum_sc(H, linear=False, nbuf=2, idx_halves=2, async_scatter=True):
    mesh = plsc.VectorSubcoreMesh(core_axis_name="c", subcore_axis_name="s")

    hc = CPT // idx_halves
    assert hc % nbuf == 0

    def body(t_hbm, src_hbm, dst_hbm, z_hbm, out_hbm, src_v, dst_v, *rest):
        rows = rest[:nbuf]
        acc_sh = rest[nbuf]
        sem_g = rest[nbuf + 1:2 * nbuf + 1]
        sem_s = rest[2 * nbuf + 1:]
        c = lax.axis_index("c")
        s = lax.axis_index("s")
        wid = c * NS + s
        pltpu.sync_copy(z_hbm.at[pl.ds(s * RPT, RPT)],
                        acc_sh.at[pl.ds(s * RPT, RPT)])
        plsc.subcore_barrier()

        for half in range(idx_halves):
            pltpu.sync_copy(src_hbm.at[pl.ds((wid * CPT + half * hc), hc)],
                            src_v)
            pltpu.sync_copy(dst_hbm.at[pl.ds((wid * CPT + half * hc), hc)],
                            dst_v)
            for b in range(nbuf - 1):
                pltpu.async_copy(t_hbm.at[src_v.at[b]], rows[b], sem_g[b])

            @pl.loop(0, hc, step=nbuf)
            def _(i):
                for b in range(nbuf):
                    j = i + b
                    nb = (b + nbuf - 1) % nbuf
                    pltpu.make_async_copy(t_hbm.at[src_v.at[j]], rows[b],
                                          sem_g[b]).wait()

                    def _prefetch():
                        pltpu.async_copy(t_hbm.at[src_v.at[j + nbuf - 1]],
                                         rows[nb], sem_g[nb])

                    if not async_scatter:
                        pltpu.sync_copy(rows[b], acc_sh.at[dst_v.at[j]],
                                        add=True)

                        @pl.when(j + nbuf - 1 < hc)
                        def _():
                            _prefetch()
                    else:
                        pltpu.async_copy(rows[b], acc_sh.at[dst_v.at[j]],
                                         sem_s[b], add=True)
                        if b == 0:
                            @pl.when((i > 0) & (j + nbuf - 1 < hc))
                            def _():
                                pltpu.make_async_copy(
                                    rows[nb], acc_sh.at[dst_v.at[j]],
                                    sem_s[nb]).wait()
                                _prefetch()

                            @pl.when((i == 0) & (j + nbuf - 1 < hc))
                            def _():
                                _prefetch()
                        else:
                            @pl.when(j + nbuf - 1 < hc)
                            def _():
                                pltpu.make_async_copy(
                                    rows[nb], acc_sh.at[dst_v.at[j]],
                                    sem_s[nb]).wait()
                                _prefetch()

            if async_scatter:
                for b in range(nbuf):
                    pltpu.make_async_copy(rows[b], acc_sh.at[dst_v.at[0]],
                                          sem_s[b]).wait()

        plsc.subcore_barrier()
        pltpu.sync_copy(acc_sh.at[pl.ds(s * RPT, RPT)],
                        out_hbm.at[pl.ds(c * NA + s * RPT, RPT)])

    scratch = (
        [pltpu.VMEM((hc, CH), jnp.int32),
         pltpu.VMEM((hc, CH), jnp.int32)]
        + [pltpu.VMEM((CH, H), jnp.float32) for _ in range(nbuf)]
        + [pltpu.VMEM_SHARED((NA, H), jnp.float32)]
        + [pltpu.SemaphoreType.DMA for _ in range(2 * nbuf)]
    )
    cp = (pltpu.CompilerParams(use_tc_tiling_on_sc=False) if linear else None)
    return pl.kernel(
        body,
        out_type=jax.ShapeDtypeStruct((NC * NA, H), jnp.float32),
        mesh=mesh,
        scratch_types=scratch,
        compiler_params=cp,
    )


def _proj_body(x_ref, w_ref, o_ref):
    o_ref[:N] = jnp.dot(x_ref[...], w_ref[...], preferred_element_type=jnp.float32)
    o_ref[N:] = jnp.zeros((PAD_ROWS, o_ref.shape[1]), jnp.float32)


def _proj(x, w):
    hp = w.shape[1]
    return pl.pallas_call(
        _proj_body,
        out_shape=jax.ShapeDtypeStruct((NP, hp), jnp.float32),
    )(x, w)


def _layer_body(hw, q_ref, x_ref, wr_ref, b_ref, g_ref, be_ref, wn_ref, t_ref, h_ref):
    a = (q_ref[:N, :hw] + q_ref[NA:NA + N, :hw]
         + jnp.dot(x_ref[...], wr_ref[...], preferred_element_type=jnp.float32)
         + b_ref[...])
    mu = jnp.mean(a, axis=0, keepdims=True)
    var = jnp.mean(jnp.square(a - mu), axis=0, keepdims=True)
    h = jnp.maximum((a - mu) / jnp.sqrt(var + EPS) * g_ref[...] + be_ref[...], 0.0)
    h_ref[...] = h
    t_ref[:N] = jnp.dot(h, wn_ref[...], preferred_element_type=jnp.float32)
    t_ref[N:] = jnp.zeros((PAD_ROWS, t_ref.shape[1]), jnp.float32)


def _layer(q, hw, x, w_root, b, g, be, w_next):
    hn = w_next.shape[1]
    return pl.pallas_call(
        functools.partial(_layer_body, hw),
        out_shape=(jax.ShapeDtypeStruct((NP, hn), jnp.float32),
                   jax.ShapeDtypeStruct((N, hw), jnp.float32)),
    )(q, x, w_root, b.reshape(1, -1), g.reshape(1, -1), be.reshape(1, -1), w_next)


def _final_body(q_ref, h_ref, wr_ref, b_ref, o_ref):
    o_ref[...] = (q_ref[:N, :OUT] + q_ref[NA:NA + N, :OUT]
                  + jnp.dot(h_ref[...], wr_ref[...],
                            preferred_element_type=jnp.float32)
                  + b_ref[...])


def _final(q, h, w_root, b):
    return pl.pallas_call(
        _final_body,
        out_shape=jax.ShapeDtypeStruct((N, OUT), jnp.float32),
    )(q, h, w_root, b.reshape(1, -1))


def kernel(x, edge_index, W1_rel, W1_root, b1, g1, be1, W2_rel, W2_root, b2,
           g2, be2, W3_rel, W3_root, b3):
    src = edge_index[0]
    dst = edge_index[1]
    pad = E_PAD - E
    pad_src = (jnp.arange(pad, dtype=jnp.int32) % PAD_ROWS) + N
    pad_dst = jnp.arange(pad, dtype=jnp.int32) % 1024
    src_p = jnp.concatenate([src, pad_src]).reshape(E_PAD // CH, CH)
    dst_p = jnp.concatenate([dst, pad_dst]).reshape(E_PAD // CH, CH)

    z128 = jnp.zeros((NA, H1), jnp.float32)
    z64 = jnp.zeros((NA, H2), jnp.float32)
    z16 = jnp.zeros((NA, OUT_PAD), jnp.float32)
    w3n = jnp.pad(W3_rel, ((0, 0), (0, OUT_PAD - OUT)))

    t1 = _proj(x, W1_rel)
    q1 = _segsum_sc(H1, async_scatter=False)(t1, src_p, dst_p, z128)
    t2, h1 = _layer(q1, H1, x, W1_root, b1, g1, be1, W2_rel)
    q2 = _segsum_sc(H2, linear=True, nbuf=4, idx_halves=1)(t2, src_p, dst_p, z64)
    t3, h2 = _layer(q2, H2, h1, W2_root, b2, g2, be2, w3n)
    q3 = _segsum_sc(OUT_PAD, linear=True, nbuf=4, idx_halves=1)(t3, src_p, dst_p, z16)
    return _final(q3, h2, W3_root, b3)


# speedup vs baseline: 1.1423x; 1.1423x over previous
import functools

import jax
import jax.numpy as jnp
from jax import lax
from jax.experimental import pallas as pl
from jax.experimental.pallas import tpu as pltpu
from jax.experimental.pallas import tpu_sc as plsc

N = 10000
E = 320000
D_IN = 128
H1 = 128
H2 = 64
OUT = 2
OUT_PAD = 16
EPS = 1e-5

NC = 2
NS = 16
NW = NC * NS
CH = 128
CPT = 80
HCPT = 40
E_PAD = CPT * NW * CH
PAD_ROWS = 240
NP = N + PAD_ROWS
SPT = NP // NS
RPT = 632
NA = NS * RPT


def _segsum_sc(H, linear=False, nbuf=2, idx_halves=2, async_scatter=True):
    mesh = plsc.VectorSubcoreMesh(core_axis_name="c", subcore_axis_name="s")

    hc = CPT // idx_halves
    assert hc % nbuf == 0

    def body(t_hbm, src_hbm, dst_hbm, z_hbm, out_hbm, src_v, dst_v, *rest):
        rows = rest[:nbuf]
        acc_sh = rest[nbuf]
        sem_g = rest[nbuf + 1:2 * nbuf + 1]
        sem_s = rest[2 * nbuf + 1:]
        c = lax.axis_index("c")
        s = lax.axis_index("s")
        wid = c * NS + s
        pltpu.sync_copy(z_hbm.at[pl.ds(s * RPT, RPT)],
                        acc_sh.at[pl.ds(s * RPT, RPT)])
        plsc.subcore_barrier()

        for half in range(idx_halves):
            pltpu.sync_copy(src_hbm.at[pl.ds((wid * CPT + half * hc), hc)],
                            src_v)
            pltpu.sync_copy(dst_hbm.at[pl.ds((wid * CPT + half * hc), hc)],
                            dst_v)
            for b in range(nbuf - 1):
                pltpu.async_copy(t_hbm.at[src_v.at[b]], rows[b], sem_g[b])

            @pl.loop(0, hc, step=nbuf)
            def _(i):
                for b in range(nbuf):
                    j = i + b
                    nb = (b + nbuf - 1) % nbuf
                    pltpu.make_async_copy(t_hbm.at[src_v.at[j]], rows[b],
                                          sem_g[b]).wait()

                    def _prefetch():
                        pltpu.async_copy(t_hbm.at[src_v.at[j + nbuf - 1]],
                                         rows[nb], sem_g[nb])

                    if not async_scatter:
                        @pl.when(j + nbuf - 1 < hc)
                        def _():
                            _prefetch()

                        pltpu.sync_copy(rows[b], acc_sh.at[dst_v.at[j]],
                                        add=True)
                    else:
                        pltpu.async_copy(rows[b], acc_sh.at[dst_v.at[j]],
                                         sem_s[b], add=True)
                        if b == 0:
                            @pl.when((i > 0) & (j + nbuf - 1 < hc))
                            def _():
                                pltpu.make_async_copy(
                                    rows[nb], acc_sh.at[dst_v.at[j]],
                                    sem_s[nb]).wait()
                                _prefetch()

                            @pl.when((i == 0) & (j + nbuf - 1 < hc))
                            def _():
                                _prefetch()
                        else:
                            @pl.when(j + nbuf - 1 < hc)
                            def _():
                                pltpu.make_async_copy(
                                    rows[nb], acc_sh.at[dst_v.at[j]],
                                    sem_s[nb]).wait()
                                _prefetch()

            if async_scatter:
                for b in range(nbuf):
                    pltpu.make_async_copy(rows[b], acc_sh.at[dst_v.at[0]],
                                          sem_s[b]).wait()

        plsc.subcore_barrier()
        pltpu.sync_copy(acc_sh.at[pl.ds(s * RPT, RPT)],
                        out_hbm.at[pl.ds(c * NA + s * RPT, RPT)])

    scratch = (
        [pltpu.VMEM((hc, CH), jnp.int32),
         pltpu.VMEM((hc, CH), jnp.int32)]
        + [pltpu.VMEM((CH, H), jnp.float32) for _ in range(nbuf)]
        + [pltpu.VMEM_SHARED((NA, H), jnp.float32)]
        + [pltpu.SemaphoreType.DMA for _ in range(2 * nbuf)]
    )
    cp = (pltpu.CompilerParams(use_tc_tiling_on_sc=False) if linear else None)
    return pl.kernel(
        body,
        out_type=jax.ShapeDtypeStruct((NC * NA, H), jnp.float32),
        mesh=mesh,
        scratch_types=scratch,
        compiler_params=cp,
    )


def _proj_body(x_ref, w_ref, o_ref):
    o_ref[:N] = jnp.dot(x_ref[...], w_ref[...], preferred_element_type=jnp.float32)
    o_ref[N:] = jnp.zeros((PAD_ROWS, o_ref.shape[1]), jnp.float32)


def _proj(x, w):
    hp = w.shape[1]
    return pl.pallas_call(
        _proj_body,
        out_shape=jax.ShapeDtypeStruct((NP, hp), jnp.float32),
    )(x, w)


def _layer_body(hw, q_ref, x_ref, wr_ref, b_ref, g_ref, be_ref, wn_ref, t_ref, h_ref):
    a = (q_ref[:N, :hw] + q_ref[NA:NA + N, :hw]
         + jnp.dot(x_ref[...], wr_ref[...], preferred_element_type=jnp.float32)
         + b_ref[...])
    mu = jnp.mean(a, axis=0, keepdims=True)
    var = jnp.mean(jnp.square(a - mu), axis=0, keepdims=True)
    h = jnp.maximum((a - mu) / jnp.sqrt(var + EPS) * g_ref[...] + be_ref[...], 0.0)
    h_ref[...] = h
    t_ref[:N] = jnp.dot(h, wn_ref[...], preferred_element_type=jnp.float32)
    t_ref[N:] = jnp.zeros((PAD_ROWS, t_ref.shape[1]), jnp.float32)


def _layer(q, hw, x, w_root, b, g, be, w_next):
    hn = w_next.shape[1]
    return pl.pallas_call(
        functools.partial(_layer_body, hw),
        out_shape=(jax.ShapeDtypeStruct((NP, hn), jnp.float32),
                   jax.ShapeDtypeStruct((N, hw), jnp.float32)),
    )(q, x, w_root, b.reshape(1, -1), g.reshape(1, -1), be.reshape(1, -1), w_next)


def _final_body(q_ref, h_ref, wr_ref, b_ref, o_ref):
    o_ref[...] = (q_ref[:N, :OUT] + q_ref[NA:NA + N, :OUT]
                  + jnp.dot(h_ref[...], wr_ref[...],
                            preferred_element_type=jnp.float32)
                  + b_ref[...])


def _final(q, h, w_root, b):
    return pl.pallas_call(
        _final_body,
        out_shape=jax.ShapeDtypeStruct((N, OUT), jnp.float32),
    )(q, h, w_root, b.reshape(1, -1))


def kernel(x, edge_index, W1_rel, W1_root, b1, g1, be1, W2_rel, W2_root, b2,
           g2, be2, W3_rel, W3_root, b3):
    src = edge_index[0]
    dst = edge_index[1]
    pad = E_PAD - E
    pad_src = (jnp.arange(pad, dtype=jnp.int32) % PAD_ROWS) + N
    pad_dst = jnp.arange(pad, dtype=jnp.int32) % 1024
    src_p = jnp.concatenate([src, pad_src]).reshape(E_PAD // CH, CH)
    dst_p = jnp.concatenate([dst, pad_dst]).reshape(E_PAD // CH, CH)

    z128 = jnp.zeros((NA, H1), jnp.float32)
    z64 = jnp.zeros((NA, H2), jnp.float32)
    z16 = jnp.zeros((NA, OUT_PAD), jnp.float32)
    w3n = jnp.pad(W3_rel, ((0, 0), (0, OUT_PAD - OUT)))

    t1 = _proj(x, W1_rel)
    q1 = _segsum_sc(H1, async_scatter=False)(t1, src_p, dst_p, z128)
    t2, h1 = _layer(q1, H1, x, W1_root, b1, g1, be1, W2_rel)
    q2 = _segsum_sc(H2, linear=True, nbuf=4, idx_halves=1)(t2, src_p, dst_p, z64)
    t3, h2 = _layer(q2, H2, h1, W2_root, b2, g2, be2, w3n)
    q3 = _segsum_sc(OUT_PAD, linear=True, nbuf=4, idx_halves=1)(t3, src_p, dst_p, z16)
    return _final(q3, h2, W3_root, b3)


# speedup vs baseline: 1.2215x; 1.0693x over previous
import functools

import jax
import jax.numpy as jnp
from jax import lax
from jax.experimental import pallas as pl
from jax.experimental.pallas import tpu as pltpu
from jax.experimental.pallas import tpu_sc as plsc

N = 10000
E = 320000
D_IN = 128
H1 = 128
H2 = 64
OUT = 2
OUT_PAD = 16
EPS = 1e-5

NC = 2
NS = 16
NW = NC * NS
CH = 128
CPT = 80
HCPT = 40
E_PAD = CPT * NW * CH
PAD_ROWS = 240
NP = N + PAD_ROWS
SPT = NP // NS
RPT = 632
NA = NS * RPT


def _segsum_sc(H, linear=False, nbuf=2, idx_halves=2, async_scatter=True):
    mesh = plsc.VectorSubcoreMesh(core_axis_name="c", subcore_axis_name="s")

    hc = CPT // idx_halves
    assert hc % nbuf == 0

    def body(t_hbm, src_hbm, dst_hbm, z_hbm, out_hbm, src_v, dst_v, *rest):
        rows = rest[:nbuf]
        acc_sh = rest[nbuf]
        sem_g = rest[nbuf + 1:2 * nbuf + 1]
        sem_s = rest[2 * nbuf + 1:]
        c = lax.axis_index("c")
        s = lax.axis_index("s")
        wid = c * NS + s
        pltpu.sync_copy(z_hbm.at[pl.ds(s * RPT, RPT)],
                        acc_sh.at[pl.ds(s * RPT, RPT)])
        plsc.subcore_barrier()

        for half in range(idx_halves):
            pltpu.sync_copy(src_hbm.at[pl.ds((wid * CPT + half * hc), hc)],
                            src_v)
            pltpu.sync_copy(dst_hbm.at[pl.ds((wid * CPT + half * hc), hc)],
                            dst_v)
            if not async_scatter:
                rows_a, rows_b = rows[0], rows[1]
                sem_a, sem_b = sem_g[0], sem_g[1]
                pltpu.async_copy(t_hbm.at[src_v.at[0]], rows_a, sem_a)

                @pl.loop(0, hc, step=2)
                def _(i):
                    pltpu.async_copy(t_hbm.at[src_v.at[i + 1]], rows_b, sem_b)
                    pltpu.make_async_copy(t_hbm.at[src_v.at[i]], rows_a,
                                          sem_a).wait()
                    pltpu.sync_copy(rows_a, acc_sh.at[dst_v.at[i]], add=True)

                    @pl.when(i + 2 < hc)
                    def _():
                        pltpu.async_copy(t_hbm.at[src_v.at[i + 2]], rows_a,
                                         sem_a)

                    pltpu.make_async_copy(t_hbm.at[src_v.at[i + 1]], rows_b,
                                          sem_b).wait()
                    pltpu.sync_copy(rows_b, acc_sh.at[dst_v.at[i + 1]],
                                    add=True)

                continue

            for b in range(nbuf - 1):
                pltpu.async_copy(t_hbm.at[src_v.at[b]], rows[b], sem_g[b])

            @pl.loop(0, hc, step=nbuf)
            def _(i):
                for b in range(nbuf):
                    j = i + b
                    nb = (b + nbuf - 1) % nbuf
                    pltpu.make_async_copy(t_hbm.at[src_v.at[j]], rows[b],
                                          sem_g[b]).wait()

                    def _prefetch():
                        pltpu.async_copy(t_hbm.at[src_v.at[j + nbuf - 1]],
                                         rows[nb], sem_g[nb])

                    if True:
                        pltpu.async_copy(rows[b], acc_sh.at[dst_v.at[j]],
                                         sem_s[b], add=True)
                        if b == 0:
                            @pl.when((i > 0) & (j + nbuf - 1 < hc))
                            def _():
                                pltpu.make_async_copy(
                                    rows[nb], acc_sh.at[dst_v.at[j]],
                                    sem_s[nb]).wait()
                                _prefetch()

                            @pl.when((i == 0) & (j + nbuf - 1 < hc))
                            def _():
                                _prefetch()
                        else:
                            @pl.when(j + nbuf - 1 < hc)
                            def _():
                                pltpu.make_async_copy(
                                    rows[nb], acc_sh.at[dst_v.at[j]],
                                    sem_s[nb]).wait()
                                _prefetch()

            if async_scatter:
                for b in range(nbuf):
                    pltpu.make_async_copy(rows[b], acc_sh.at[dst_v.at[0]],
                                          sem_s[b]).wait()

        plsc.subcore_barrier()
        pltpu.sync_copy(acc_sh.at[pl.ds(s * RPT, RPT)],
                        out_hbm.at[pl.ds(c * NA + s * RPT, RPT)])

    scratch = (
        [pltpu.VMEM((hc, CH), jnp.int32),
         pltpu.VMEM((hc, CH), jnp.int32)]
        + [pltpu.VMEM((CH, H), jnp.float32) for _ in range(nbuf)]
        + [pltpu.VMEM_SHARED((NA, H), jnp.float32)]
        + [pltpu.SemaphoreType.DMA for _ in range(2 * nbuf)]
    )
    cp = (pltpu.CompilerParams(use_tc_tiling_on_sc=False) if linear else None)
    return pl.kernel(
        body,
        out_type=jax.ShapeDtypeStruct((NC * NA, H), jnp.float32),
        mesh=mesh,
        scratch_types=scratch,
        compiler_params=cp,
    )


def _proj_body(x_ref, w_ref, o_ref):
    o_ref[:N] = jnp.dot(x_ref[...], w_ref[...], preferred_element_type=jnp.float32)
    o_ref[N:] = jnp.zeros((PAD_ROWS, o_ref.shape[1]), jnp.float32)


def _proj(x, w):
    hp = w.shape[1]
    return pl.pallas_call(
        _proj_body,
        out_shape=jax.ShapeDtypeStruct((NP, hp), jnp.float32),
    )(x, w)


def _layer_body(hw, q_ref, x_ref, wr_ref, b_ref, g_ref, be_ref, wn_ref, t_ref, h_ref):
    a = (q_ref[:N, :hw] + q_ref[NA:NA + N, :hw]
         + jnp.dot(x_ref[...], wr_ref[...], preferred_element_type=jnp.float32)
         + b_ref[...])
    mu = jnp.mean(a, axis=0, keepdims=True)
    var = jnp.mean(jnp.square(a - mu), axis=0, keepdims=True)
    h = jnp.maximum((a - mu) / jnp.sqrt(var + EPS) * g_ref[...] + be_ref[...], 0.0)
    h_ref[...] = h
    t_ref[:N] = jnp.dot(h, wn_ref[...], preferred_element_type=jnp.float32)
    t_ref[N:] = jnp.zeros((PAD_ROWS, t_ref.shape[1]), jnp.float32)


def _layer(q, hw, x, w_root, b, g, be, w_next):
    hn = w_next.shape[1]
    return pl.pallas_call(
        functools.partial(_layer_body, hw),
        out_shape=(jax.ShapeDtypeStruct((NP, hn), jnp.float32),
                   jax.ShapeDtypeStruct((N, hw), jnp.float32)),
    )(q, x, w_root, b.reshape(1, -1), g.reshape(1, -1), be.reshape(1, -1), w_next)


def _final_body(q_ref, h_ref, wr_ref, b_ref, o_ref):
    o_ref[...] = (q_ref[:N, :OUT] + q_ref[NA:NA + N, :OUT]
                  + jnp.dot(h_ref[...], wr_ref[...],
                            preferred_element_type=jnp.float32)
                  + b_ref[...])


def _final(q, h, w_root, b):
    return pl.pallas_call(
        _final_body,
        out_shape=jax.ShapeDtypeStruct((N, OUT), jnp.float32),
    )(q, h, w_root, b.reshape(1, -1))


def kernel(x, edge_index, W1_rel, W1_root, b1, g1, be1, W2_rel, W2_root, b2,
           g2, be2, W3_rel, W3_root, b3):
    src = edge_index[0]
    dst = edge_index[1]
    pad = E_PAD - E
    pad_src = (jnp.arange(pad, dtype=jnp.int32) % PAD_ROWS) + N
    pad_dst = jnp.arange(pad, dtype=jnp.int32) % 1024
    src_p = jnp.concatenate([src, pad_src]).reshape(E_PAD // CH, CH)
    dst_p = jnp.concatenate([dst, pad_dst]).reshape(E_PAD // CH, CH)

    z128 = jnp.zeros((NA, H1), jnp.float32)
    z64 = jnp.zeros((NA, H2), jnp.float32)
    z16 = jnp.zeros((NA, OUT_PAD), jnp.float32)
    w3n = jnp.pad(W3_rel, ((0, 0), (0, OUT_PAD - OUT)))

    t1 = _proj(x, W1_rel)
    q1 = _segsum_sc(H1, async_scatter=False)(t1, src_p, dst_p, z128)
    t2, h1 = _layer(q1, H1, x, W1_root, b1, g1, be1, W2_rel)
    q2 = _segsum_sc(H2, linear=True, nbuf=4, idx_halves=1)(t2, src_p, dst_p, z64)
    t3, h2 = _layer(q2, H2, h1, W2_root, b2, g2, be2, w3n)
    q3 = _segsum_sc(OUT_PAD, linear=True, nbuf=4, idx_halves=1)(t3, src_p, dst_p, z16)
    return _final(q3, h2, W3_root, b3)
